# baseline (speedup 1.0000x reference)
import functools

import jax
import jax.numpy as jnp
from jax import lax
from jax.experimental import pallas as pl
from jax.experimental.pallas import tpu as pltpu
from jax.experimental.pallas import tpu_sc as plsc

CHUNK = 512
NBUF = 4
L = 16


@functools.lru_cache(maxsize=None)
def _make(B: int, D: int, V: int):
    info = plsc.get_sparse_core_info()
    NC, NS = info.num_cores, info.num_subcores
    NW = NC * NS
    assert B % (NW * CHUNK * NBUF) == 0
    b_per_w = B // NW
    n_groups = b_per_w // (CHUNK * NBUF)
    mesh = plsc.VectorSubcoreMesh(core_axis_name="c", subcore_axis_name="s")

    scratch = (
        [pltpu.VMEM((CHUNK,), jnp.int32) for _ in range(NBUF)]
        + [pltpu.VMEM((CHUNK, D), jnp.float32) for _ in range(NBUF)]
        + [pltpu.SemaphoreType.DMA for _ in range(NBUF)]
        + [pltpu.VMEM((V, D), jnp.float32)]
    )

    @functools.partial(
        pl.kernel,
        mesh=mesh,
        compiler_params=pltpu.CompilerParams(
            use_tc_tiling_on_sc=False, needs_layout_passes=False
        ),
        out_type=jax.ShapeDtypeStruct((B, D), jnp.float32),
        scratch_types=scratch,
    )
    def k(idx_hbm, table_hbm, out_hbm, *scr):
        idx_vs = scr[:NBUF]
        rows_vs = scr[NBUF : 2 * NBUF]
        osems = scr[2 * NBUF : 3 * NBUF]
        table_v = scr[3 * NBUF]
        wid = lax.axis_index("s") * NC + lax.axis_index("c")
        base = wid * b_per_w

        pltpu.sync_copy(table_hbm, table_v)
        lane = lax.iota(jnp.int32, L)

        def group(gi, carry):
            for b in range(NBUF):
                off = base + (gi * NBUF + b) * CHUNK

                @pl.when(gi > 0)
                def _drain(b=b, off=off):
                    pltpu.make_async_copy(
                        rows_vs[b], out_hbm.at[pl.ds(off, CHUNK)], osems[b]
                    ).wait()

                pltpu.sync_copy(idx_hbm.at[pl.ds(off, CHUNK)], idx_vs[b])

                def expand(j, c2, b=b):
                    idx16 = idx_vs[b][pl.ds(j * L, L)]
                    row16 = j * L + lane
                    for c in range(D):
                        col = jnp.full((L,), c, jnp.int32)
                        v = plsc.load_gather(table_v, [idx16, col])
                        plsc.store_scatter(rows_vs[b], [row16, col], v)
                    return c2

                lax.fori_loop(0, CHUNK // L, expand, 0)
                pltpu.async_copy(
                    rows_vs[b], out_hbm.at[pl.ds(off, CHUNK)], osems[b]
                )
            return carry

        lax.fori_loop(0, n_groups, group, 0)
        for b in range(NBUF):
            pltpu.make_async_copy(
                rows_vs[b],
                out_hbm.at[pl.ds(base + b * CHUNK, CHUNK)],
                osems[b],
            ).wait()

    return k


def kernel(x, table):
    B0, H = x.shape
    D = table.shape[1]
    idx = x.reshape(B0 * H).astype(jnp.int32)
    out = _make(B0 * H, D, table.shape[0])(idx, table)
    return out.reshape(B0, H, D)


# speedup vs baseline: 2.4430x; 2.4430x over previous
import functools

import jax
import jax.numpy as jnp
from jax import lax
from jax.experimental import pallas as pl
from jax.experimental.pallas import tpu as pltpu
from jax.experimental.pallas import tpu_sc as plsc

CHUNK = 512
NBUF = 4


@functools.lru_cache(maxsize=None)
def _make(B: int, D: int, V: int):
    info = plsc.get_sparse_core_info()
    NC, NS = info.num_cores, info.num_subcores
    NW = NC * NS
    assert B % (NW * CHUNK * NBUF) == 0
    b_per_w = B // NW
    n_groups = b_per_w // (CHUNK * NBUF)
    mesh = plsc.VectorSubcoreMesh(core_axis_name="c", subcore_axis_name="s")

    scratch = (
        [pltpu.VMEM((CHUNK,), jnp.int32) for _ in range(NBUF)]
        + [pltpu.VMEM((CHUNK, D), jnp.float32) for _ in range(NBUF)]
        + [pltpu.SemaphoreType.DMA for _ in range(2 * NBUF)]
        + [pltpu.VMEM_SHARED((V, D), jnp.float32)]
    )

    @functools.partial(
        pl.kernel,
        mesh=mesh,
        compiler_params=pltpu.CompilerParams(use_tc_tiling_on_sc=False),
        out_type=jax.ShapeDtypeStruct((B, D), jnp.float32),
        scratch_types=scratch,
    )
    def k(idx_hbm, table_hbm, out_hbm, *scr):
        idx_vs = scr[:NBUF]
        rows_vs = scr[NBUF : 2 * NBUF]
        gsems = scr[2 * NBUF : 3 * NBUF]
        osems = scr[3 * NBUF : 4 * NBUF]
        table_sh = scr[4 * NBUF]
        sid = lax.axis_index("s")
        wid = sid * NC + lax.axis_index("c")
        base = wid * b_per_w

        @pl.when(sid == 0)
        def _stage():
            pltpu.sync_copy(table_hbm, table_sh)

        plsc.subcore_barrier()

        def group(gi, carry):
            offs = [base + (gi * NBUF + b) * CHUNK for b in range(NBUF)]
            gathers = []
            for b in range(NBUF):
                @pl.when(gi > 0)
                def _drain(b=b):
                    pltpu.make_async_copy(
                        rows_vs[b], out_hbm.at[pl.ds(offs[b], CHUNK)], osems[b]
                    ).wait()

                pltpu.sync_copy(idx_hbm.at[pl.ds(offs[b], CHUNK)], idx_vs[b])
                src = table_sh if b % 2 == 0 else table_hbm
                gathers.append(
                    pltpu.async_copy(src.at[idx_vs[b]], rows_vs[b], gsems[b])
                )
            for b in range(NBUF):
                gathers[b].wait()
                pltpu.async_copy(
                    rows_vs[b], out_hbm.at[pl.ds(offs[b], CHUNK)], osems[b]
                )
            return carry

        lax.fori_loop(0, n_groups, group, 0)
        for b in range(NBUF):
            pltpu.make_async_copy(
                rows_vs[b], out_hbm.at[pl.ds(base + b * CHUNK, CHUNK)], osems[b]
            ).wait()

    return k


def kernel(x, table):
    B0, H = x.shape
    D = table.shape[1]
    idx = x.reshape(B0 * H).astype(jnp.int32)
    out = _make(B0 * H, D, table.shape[0])(idx, table)
    return out.reshape(B0, H, D)


# speedup vs baseline: 2.6582x; 1.0881x over previous
import functools

import jax
import jax.numpy as jnp
from jax import lax
from jax.experimental import pallas as pl
from jax.experimental.pallas import tpu as pltpu
from jax.experimental.pallas import tpu_sc as plsc

CHUNK = 512
NBUF = 4


@functools.lru_cache(maxsize=None)
def _make(B: int, D: int, V: int):
    info = plsc.get_sparse_core_info()
    NC, NS = info.num_cores, info.num_subcores
    NW = NC * NS
    assert B % (NW * CHUNK * NBUF) == 0
    b_per_w = B // NW
    n_groups = b_per_w // (CHUNK * NBUF)
    mesh = plsc.VectorSubcoreMesh(core_axis_name="c", subcore_axis_name="s")

    scratch = (
        [pltpu.VMEM((CHUNK,), jnp.int32) for _ in range(NBUF)]
        + [pltpu.VMEM((CHUNK, D), jnp.float32) for _ in range(NBUF)]
        + [pltpu.SemaphoreType.DMA for _ in range(2 * NBUF)]
        + [pltpu.VMEM_SHARED((V, D), jnp.float32)]
    )

    @functools.partial(
        pl.kernel,
        mesh=mesh,
        compiler_params=pltpu.CompilerParams(use_tc_tiling_on_sc=False),
        out_type=jax.ShapeDtypeStruct((B, D), jnp.float32),
        scratch_types=scratch,
    )
    def k(idx_hbm, table_hbm, out_hbm, *scr):
        idx_vs = scr[:NBUF]
        rows_vs = scr[NBUF : 2 * NBUF]
        gsems = scr[2 * NBUF : 3 * NBUF]
        osems = scr[3 * NBUF : 4 * NBUF]
        table_sh = scr[4 * NBUF]
        sid = lax.axis_index("s")
        wid = sid * NC + lax.axis_index("c")
        base = wid * b_per_w

        @pl.when(sid == 0)
        def _stage():
            pltpu.sync_copy(table_hbm, table_sh)

        plsc.subcore_barrier()

        def group(gi, carry):
            offs = [base + (gi * NBUF + b) * CHUNK for b in range(NBUF)]
            gathers = []
            for b in range(NBUF):
                @pl.when(gi > 0)
                def _drain(b=b):
                    pltpu.make_async_copy(
                        rows_vs[b], out_hbm.at[pl.ds(offs[b], CHUNK)], osems[b]
                    ).wait()

                pltpu.sync_copy(idx_hbm.at[pl.ds(offs[b], CHUNK)], idx_vs[b])
                src = table_hbm if b == NBUF - 1 else table_sh
                gathers.append(
                    pltpu.async_copy(src.at[idx_vs[b]], rows_vs[b], gsems[b])
                )
            for b in range(NBUF):
                gathers[b].wait()
                pltpu.async_copy(
                    rows_vs[b], out_hbm.at[pl.ds(offs[b], CHUNK)], osems[b]
                )
            return carry

        lax.fori_loop(0, n_groups, group, 0)
        for b in range(NBUF):
            pltpu.make_async_copy(
                rows_vs[b], out_hbm.at[pl.ds(base + b * CHUNK, CHUNK)], osems[b]
            ).wait()

    return k


def kernel(x, table):
    B0, H = x.shape
    D = table.shape[1]
    idx = x.reshape(B0 * H).astype(jnp.int32)
    out = _make(B0 * H, D, table.shape[0])(idx, table)
    return out.reshape(B0, H, D)


# speedup vs baseline: 2.9082x; 1.0940x over previous
import functools

import jax
import jax.numpy as jnp
from jax import lax
from jax.experimental import pallas as pl
from jax.experimental.pallas import tpu as pltpu
from jax.experimental.pallas import tpu_sc as plsc

CHUNK = 1600
NBUF = 2


@functools.lru_cache(maxsize=None)
def _make(B: int, D: int, V: int):
    info = plsc.get_sparse_core_info()
    NC, NS = info.num_cores, info.num_subcores
    NW = NC * NS
    assert B % (NW * CHUNK * NBUF) == 0
    b_per_w = B // NW
    n_groups = b_per_w // (CHUNK * NBUF)
    mesh = plsc.VectorSubcoreMesh(core_axis_name="c", subcore_axis_name="s")

    scratch = (
        [pltpu.VMEM((CHUNK,), jnp.int32) for _ in range(NBUF)]
        + [pltpu.VMEM((CHUNK, D), jnp.float32) for _ in range(NBUF)]
        + [pltpu.SemaphoreType.DMA for _ in range(2 * NBUF)]
        + [pltpu.VMEM_SHARED((V, D), jnp.float32)]
    )

    @functools.partial(
        pl.kernel,
        mesh=mesh,
        compiler_params=pltpu.CompilerParams(use_tc_tiling_on_sc=False),
        out_type=jax.ShapeDtypeStruct((B, D), jnp.float32),
        scratch_types=scratch,
    )
    def k(idx_hbm, table_hbm, out_hbm, *scr):
        idx_vs = scr[:NBUF]
        rows_vs = scr[NBUF : 2 * NBUF]
        gsems = scr[2 * NBUF : 3 * NBUF]
        osems = scr[3 * NBUF : 4 * NBUF]
        table_sh = scr[4 * NBUF]
        sid = lax.axis_index("s")
        wid = sid * NC + lax.axis_index("c")
        base = wid * b_per_w

        @pl.when(sid == 0)
        def _stage():
            pltpu.sync_copy(table_hbm, table_sh)

        plsc.subcore_barrier()

        def group(gi, carry):
            offs = [base + (gi * NBUF + b) * CHUNK for b in range(NBUF)]
            gathers = []
            for b in range(NBUF):
                @pl.when(gi > 0)
                def _drain(b=b):
                    pltpu.make_async_copy(
                        rows_vs[b], out_hbm.at[pl.ds(offs[b], CHUNK)], osems[b]
                    ).wait()

                pltpu.sync_copy(idx_hbm.at[pl.ds(offs[b], CHUNK)], idx_vs[b])
                src = table_sh
                gathers.append(
                    pltpu.async_copy(src.at[idx_vs[b]], rows_vs[b], gsems[b])
                )
            for b in range(NBUF):
                gathers[b].wait()
                pltpu.async_copy(
                    rows_vs[b], out_hbm.at[pl.ds(offs[b], CHUNK)], osems[b]
                )
            return carry

        lax.fori_loop(0, n_groups, group, 0)
        for b in range(NBUF):
            pltpu.make_async_copy(
                rows_vs[b], out_hbm.at[pl.ds(base + b * CHUNK, CHUNK)], osems[b]
            ).wait()

    return k


def kernel(x, table):
    B0, H = x.shape
    D = table.shape[1]
    idx = x.reshape(B0 * H).astype(jnp.int32)
    out = _make(B0 * H, D, table.shape[0])(idx, table)
    return out.reshape(B0, H, D)
